# baseline (speedup 1.0000x reference)
import functools

import jax
import jax.numpy as jnp
from jax import lax
from jax.experimental import pallas as pl
from jax.experimental.pallas import tpu as pltpu
from jax.experimental.pallas import tpu_sc as plsc

EMBED = 128
HIDDEN = 1024
BATCH = 16384
NC = 2
NS = 16
NW = NC * NS
B_PER_W = BATCH // NW
CHUNK = 128
NCHUNK = B_PER_W // CHUNK


def _gather_body(xp, xo, xs, wp, wo, ws, op, oo, osub, idx_v, rows_v, sem):
    wid = lax.axis_index("s") * NC + lax.axis_index("c")
    base = wid * B_PER_W
    for x_hbm, t_hbm, o_hbm in ((xp, wp, op), (xo, wo, oo), (xs, ws, osub)):
        pltpu.sync_copy(x_hbm.at[pl.ds(base, B_PER_W)], idx_v)
        for j in range(NCHUNK):
            pltpu.async_copy(
                t_hbm.at[idx_v.at[pl.ds(j * CHUNK, CHUNK)]],
                rows_v.at[pl.ds(j * CHUNK, CHUNK)],
                sem,
            )
        for j in range(NCHUNK):
            pltpu.make_async_copy(
                t_hbm.at[idx_v.at[pl.ds(j * CHUNK, CHUNK)]],
                rows_v.at[pl.ds(j * CHUNK, CHUNK)],
                sem,
            ).wait()
        pltpu.sync_copy(rows_v, o_hbm.at[pl.ds(base, B_PER_W)])


_h_type = jax.ShapeDtypeStruct((BATCH, EMBED), jnp.float32)

_gather = pl.kernel(
    _gather_body,
    mesh=plsc.VectorSubcoreMesh(core_axis_name="c", subcore_axis_name="s"),
    out_type=(_h_type, _h_type, _h_type),
    scratch_types=[
        pltpu.VMEM((B_PER_W,), jnp.int32),
        pltpu.VMEM((B_PER_W, EMBED), jnp.float32),
        pltpu.SemaphoreType.DMA,
    ],
)


BM = 512


def _mlp_body(hp, ho, hs, w, b, o):
    h = jnp.concatenate(
        (
            jnp.maximum(hp[...], 0.0),
            jnp.maximum(ho[...], 0.0),
            jnp.maximum(hs[...], 0.0),
        ),
        axis=1,
    ).astype(jnp.bfloat16)
    acc = jnp.dot(h, w[...], preferred_element_type=jnp.float32)
    o[...] = jnp.maximum(acc + b[...], 0.0)


def _mlp(hp, ho, hs, w, b):
    grid = BATCH // BM
    return pl.pallas_call(
        _mlp_body,
        grid=(grid,),
        in_specs=[
            pl.BlockSpec((BM, EMBED), lambda i: (i, 0)),
            pl.BlockSpec((BM, EMBED), lambda i: (i, 0)),
            pl.BlockSpec((BM, EMBED), lambda i: (i, 0)),
            pl.BlockSpec((3 * EMBED, HIDDEN), lambda i: (0, 0)),
            pl.BlockSpec((1, HIDDEN), lambda i: (0, 0)),
        ],
        out_specs=pl.BlockSpec((BM, HIDDEN), lambda i: (i, 0)),
        out_shape=jax.ShapeDtypeStruct((BATCH, HIDDEN), jnp.float32),
    )(hp, ho, hs, w, b)


def kernel(X_phase, X_occurrence, X_subject, X_lengths,
           W_phase, W_occurrence, W_subject, W_out, b_out):
    del X_lengths
    hp, ho, hs = _gather(
        X_phase.astype(jnp.int32),
        X_occurrence.astype(jnp.int32),
        X_subject.astype(jnp.int32),
        W_phase, W_occurrence, W_subject,
    )
    return _mlp(hp, ho, hs, W_out.astype(jnp.bfloat16), b_out.reshape(1, HIDDEN))


# speedup vs baseline: 1.1207x; 1.1207x over previous
import functools

import jax
import jax.numpy as jnp
from jax import lax
from jax.experimental import pallas as pl
from jax.experimental.pallas import tpu as pltpu
from jax.experimental.pallas import tpu_sc as plsc

EMBED = 128
HIDDEN = 1024
BATCH = 16384
NC = 2
NS = 16
NW = NC * NS
B_PER_W = BATCH // NW
CHUNK = 128
NCHUNK = B_PER_W // CHUNK


def _gather_body(xp, xo, xs, wp, wo, ws, op, oo, osub, idx_v, rows_v, sem):
    wid = lax.axis_index("s") * NC + lax.axis_index("c")
    base = wid * B_PER_W
    for x_hbm, t_hbm, o_hbm in ((xp, wp, op), (xo, wo, oo), (xs, ws, osub)):
        pltpu.sync_copy(x_hbm.at[pl.ds(base, B_PER_W)], idx_v)
        for j in range(NCHUNK):
            pltpu.async_copy(
                t_hbm.at[idx_v.at[pl.ds(j * CHUNK, CHUNK)]],
                rows_v.at[pl.ds(j * CHUNK, CHUNK)],
                sem,
            )
        for j in range(NCHUNK):
            pltpu.make_async_copy(
                t_hbm.at[idx_v.at[pl.ds(j * CHUNK, CHUNK)]],
                rows_v.at[pl.ds(j * CHUNK, CHUNK)],
                sem,
            ).wait()
        pltpu.sync_copy(rows_v, o_hbm.at[pl.ds(base, B_PER_W)])


_h_type = jax.ShapeDtypeStruct((BATCH, EMBED), jnp.float32)

_gather = pl.kernel(
    _gather_body,
    mesh=plsc.VectorSubcoreMesh(core_axis_name="c", subcore_axis_name="s"),
    out_type=(_h_type, _h_type, _h_type),
    scratch_types=[
        pltpu.VMEM((B_PER_W,), jnp.int32),
        pltpu.VMEM((B_PER_W, EMBED), jnp.float32),
        pltpu.SemaphoreType.DMA,
    ],
)


BM = 1024


def _mlp_body(hp, ho, hs, w, b, o):
    h = jnp.concatenate(
        (
            jnp.maximum(hp[...], 0.0),
            jnp.maximum(ho[...], 0.0),
            jnp.maximum(hs[...], 0.0),
        ),
        axis=1,
    ).astype(jnp.bfloat16)
    acc = jnp.dot(h, w[...], preferred_element_type=jnp.float32)
    o[...] = jnp.maximum(acc + b[...], 0.0)


def _mlp(hp, ho, hs, w, b):
    grid = BATCH // BM
    return pl.pallas_call(
        _mlp_body,
        grid=(grid,),
        in_specs=[
            pl.BlockSpec((BM, EMBED), lambda i: (i, 0)),
            pl.BlockSpec((BM, EMBED), lambda i: (i, 0)),
            pl.BlockSpec((BM, EMBED), lambda i: (i, 0)),
            pl.BlockSpec((3 * EMBED, HIDDEN), lambda i: (0, 0)),
            pl.BlockSpec((1, HIDDEN), lambda i: (0, 0)),
        ],
        out_specs=pl.BlockSpec((BM, HIDDEN), lambda i: (i, 0)),
        out_shape=jax.ShapeDtypeStruct((BATCH, HIDDEN), jnp.float32),
    )(hp, ho, hs, w, b)


def kernel(X_phase, X_occurrence, X_subject, X_lengths,
           W_phase, W_occurrence, W_subject, W_out, b_out):
    del X_lengths
    hp, ho, hs = _gather(
        X_phase.astype(jnp.int32),
        X_occurrence.astype(jnp.int32),
        X_subject.astype(jnp.int32),
        W_phase, W_occurrence, W_subject,
    )
    return _mlp(hp, ho, hs, W_out.astype(jnp.bfloat16), b_out.reshape(1, HIDDEN))


# speedup vs baseline: 1.1581x; 1.0334x over previous
import functools

import jax
import jax.numpy as jnp
from jax import lax
from jax.experimental import pallas as pl
from jax.experimental.pallas import tpu as pltpu
from jax.experimental.pallas import tpu_sc as plsc

EMBED = 128
HIDDEN = 1024
BATCH = 16384
NC = 2
NS = 16
NW = NC * NS
B_PER_W = BATCH // NW
CHUNK = 128
NCHUNK = B_PER_W // CHUNK


def _gather_body(xp, xo, xs, wp, wo, ws, op, oo, osub, idx_v, rows_v, sem):
    wid = lax.axis_index("s") * NC + lax.axis_index("c")
    base = wid * B_PER_W
    for x_hbm, t_hbm, o_hbm in ((xp, wp, op), (xo, wo, oo), (xs, ws, osub)):
        pltpu.sync_copy(x_hbm.at[pl.ds(base, B_PER_W)], idx_v)
        for j in range(NCHUNK):
            pltpu.async_copy(
                t_hbm.at[idx_v.at[pl.ds(j * CHUNK, CHUNK)]],
                rows_v.at[pl.ds(j * CHUNK, CHUNK)],
                sem,
            )
        for j in range(NCHUNK):
            pltpu.make_async_copy(
                t_hbm.at[idx_v.at[pl.ds(j * CHUNK, CHUNK)]],
                rows_v.at[pl.ds(j * CHUNK, CHUNK)],
                sem,
            ).wait()
        pltpu.sync_copy(rows_v, o_hbm.at[pl.ds(base, B_PER_W)])


_h_type = jax.ShapeDtypeStruct((BATCH, EMBED), jnp.float32)

_gather = pl.kernel(
    _gather_body,
    mesh=plsc.VectorSubcoreMesh(core_axis_name="c", subcore_axis_name="s"),
    out_type=(_h_type, _h_type, _h_type),
    scratch_types=[
        pltpu.VMEM((B_PER_W,), jnp.int32),
        pltpu.VMEM((B_PER_W, EMBED), jnp.float32),
        pltpu.SemaphoreType.DMA,
    ],
)


BM = 2048


def _mlp_body(hp, ho, hs, w, b, o):
    h = jnp.concatenate(
        (
            jnp.maximum(hp[...], 0.0),
            jnp.maximum(ho[...], 0.0),
            jnp.maximum(hs[...], 0.0),
        ),
        axis=1,
    ).astype(jnp.bfloat16)
    acc = jnp.dot(h, w[...], preferred_element_type=jnp.float32)
    o[...] = jnp.maximum(acc + b[...], 0.0)


def _mlp(hp, ho, hs, w, b):
    grid = BATCH // BM
    return pl.pallas_call(
        _mlp_body,
        grid=(grid,),
        in_specs=[
            pl.BlockSpec((BM, EMBED), lambda i: (i, 0)),
            pl.BlockSpec((BM, EMBED), lambda i: (i, 0)),
            pl.BlockSpec((BM, EMBED), lambda i: (i, 0)),
            pl.BlockSpec((3 * EMBED, HIDDEN), lambda i: (0, 0)),
            pl.BlockSpec((1, HIDDEN), lambda i: (0, 0)),
        ],
        out_specs=pl.BlockSpec((BM, HIDDEN), lambda i: (i, 0)),
        out_shape=jax.ShapeDtypeStruct((BATCH, HIDDEN), jnp.float32),
    )(hp, ho, hs, w, b)


def kernel(X_phase, X_occurrence, X_subject, X_lengths,
           W_phase, W_occurrence, W_subject, W_out, b_out):
    del X_lengths
    hp, ho, hs = _gather(
        X_phase.astype(jnp.int32),
        X_occurrence.astype(jnp.int32),
        X_subject.astype(jnp.int32),
        W_phase, W_occurrence, W_subject,
    )
    return _mlp(hp, ho, hs, W_out.astype(jnp.bfloat16), b_out.reshape(1, HIDDEN))


# speedup vs baseline: 1.1695x; 1.0098x over previous
import functools

import jax
import jax.numpy as jnp
from jax import lax
from jax.experimental import pallas as pl
from jax.experimental.pallas import tpu as pltpu
from jax.experimental.pallas import tpu_sc as plsc

EMBED = 128
HIDDEN = 1024
BATCH = 16384
NC = 2
NS = 16
NW = NC * NS
B_PER_W = BATCH // NW
CHUNK = 128
NCHUNK = B_PER_W // CHUNK


def _gather_body(xp, xo, xs, wp, wo, ws, op, oo, osub, idx_v, rows_v, sem):
    wid = lax.axis_index("s") * NC + lax.axis_index("c")
    base = wid * B_PER_W
    for x_hbm, t_hbm, o_hbm in ((xp, wp, op), (xo, wo, oo), (xs, ws, osub)):
        pltpu.sync_copy(x_hbm.at[pl.ds(base, B_PER_W)], idx_v)
        for j in range(NCHUNK):
            pltpu.async_copy(
                t_hbm.at[idx_v.at[pl.ds(j * CHUNK, CHUNK)]],
                rows_v.at[pl.ds(j * CHUNK, CHUNK)],
                sem,
            )
        for j in range(NCHUNK):
            pltpu.make_async_copy(
                t_hbm.at[idx_v.at[pl.ds(j * CHUNK, CHUNK)]],
                rows_v.at[pl.ds(j * CHUNK, CHUNK)],
                sem,
            ).wait()
        pltpu.sync_copy(rows_v, o_hbm.at[pl.ds(base, B_PER_W)])


_h_type = jax.ShapeDtypeStruct((BATCH, EMBED), jnp.float32)

_gather = pl.kernel(
    _gather_body,
    mesh=plsc.VectorSubcoreMesh(core_axis_name="c", subcore_axis_name="s"),
    out_type=(_h_type, _h_type, _h_type),
    scratch_types=[
        pltpu.VMEM((B_PER_W,), jnp.int32),
        pltpu.VMEM((B_PER_W, EMBED), jnp.float32),
        pltpu.SemaphoreType.DMA,
    ],
)


BM = 4096


def _mlp_body(hp, ho, hs, w, b, o):
    h = jnp.concatenate(
        (
            jnp.maximum(hp[...], 0.0),
            jnp.maximum(ho[...], 0.0),
            jnp.maximum(hs[...], 0.0),
        ),
        axis=1,
    ).astype(jnp.bfloat16)
    acc = jnp.dot(h, w[...], preferred_element_type=jnp.float32)
    o[...] = jnp.maximum(acc + b[...], 0.0)


def _mlp(hp, ho, hs, w, b):
    grid = BATCH // BM
    return pl.pallas_call(
        _mlp_body,
        grid=(grid,),
        in_specs=[
            pl.BlockSpec((BM, EMBED), lambda i: (i, 0)),
            pl.BlockSpec((BM, EMBED), lambda i: (i, 0)),
            pl.BlockSpec((BM, EMBED), lambda i: (i, 0)),
            pl.BlockSpec((3 * EMBED, HIDDEN), lambda i: (0, 0)),
            pl.BlockSpec((1, HIDDEN), lambda i: (0, 0)),
        ],
        out_specs=pl.BlockSpec((BM, HIDDEN), lambda i: (i, 0)),
        out_shape=jax.ShapeDtypeStruct((BATCH, HIDDEN), jnp.float32),
    )(hp, ho, hs, w, b)


def kernel(X_phase, X_occurrence, X_subject, X_lengths,
           W_phase, W_occurrence, W_subject, W_out, b_out):
    del X_lengths
    hp, ho, hs = _gather(
        X_phase.astype(jnp.int32),
        X_occurrence.astype(jnp.int32),
        X_subject.astype(jnp.int32),
        W_phase, W_occurrence, W_subject,
    )
    return _mlp(hp, ho, hs, W_out.astype(jnp.bfloat16), b_out.reshape(1, HIDDEN))
